# baseline (speedup 1.0000x reference)
import functools

import jax
import jax.numpy as jnp
from jax import lax
from jax.experimental import pallas as pl
from jax.experimental.pallas import tpu as pltpu
from jax.experimental.pallas import tpu_sc as plsc

BATCH = 4096
HIST = 200
FEAT = 32

NUM_CORES = 2
NUM_SUBCORES = 16
NW = NUM_CORES * NUM_SUBCORES
B_PER_W = BATCH // NW
LANES = 16
FTILE = 8
NFT = FEAT // FTILE
T_CHUNK = 4
NCHUNK = HIST // T_CHUNK
NBUF = 3
KG = B_PER_W // LANES


def _embed_kernel(idx_hbm, table_hbm, out_hbm, idx_tv, bufs, obufs, gsem, osem):
    wid = lax.axis_index("s") * NUM_CORES + lax.axis_index("c")
    b0 = wid * B_PER_W
    pltpu.sync_copy(idx_hbm.at[:, pl.ds(b0, B_PER_W)], idx_tv)

    lane_iota = lax.iota(jnp.int32, LANES)

    def gather(c, tt, s):
        return pltpu.make_async_copy(
            table_hbm.at[idx_tv.at[c * T_CHUNK + tt]],
            bufs.at[s].at[pl.ds(tt * B_PER_W, B_PER_W)],
            gsem.at[s],
        )

    def out_copy(c, s):
        return pltpu.make_async_copy(
            obufs.at[s], out_hbm.at[pl.ds(c * T_CHUNK, T_CHUNK), :, wid], osem.at[s]
        )

    for c in range(NBUF - 1):
        for tt in range(T_CHUNK):
            gather(c, tt, c).start()

    def body(c, carry):
        s = c % NBUF
        s_ids = jnp.full((LANES,), s, jnp.int32)

        @pl.when(c + 2 < NCHUNK)
        def _():
            for tt in range(T_CHUNK):
                gather(c + 2, tt, (c + 2) % NBUF).start()

        for tt in range(T_CHUNK):
            gather(c, tt, s).wait()

        @pl.when(c >= NBUF)
        def _():
            out_copy(c - NBUF, s).wait()

        @plsc.parallel_loop(0, T_CHUNK * FEAT * KG, unroll=8)
        def _(i):
            tt = i // (FEAT * KG)
            rem = i % (FEAT * KG)
            f = rem // KG
            k = rem % KG
            v = plsc.load_gather(
                bufs,
                [
                    s_ids,
                    tt * B_PER_W + k * LANES + lane_iota,
                    jnp.full((LANES,), f, jnp.int32),
                ],
            )
            obufs[s, tt, f // FTILE, f % FTILE, pl.ds(k * LANES, LANES)] = v

        out_copy(c, s).start()
        return carry

    lax.fori_loop(0, NCHUNK, body, None)
    for c in range(NCHUNK - NBUF, NCHUNK):
        out_copy(c, c % NBUF).wait()


@jax.jit
def _embed(idx_t, table):
    k = functools.partial(
        pl.kernel,
        mesh=plsc.VectorSubcoreMesh(core_axis_name="c", subcore_axis_name="s"),
        out_type=jax.ShapeDtypeStruct((HIST, NFT, NW, FTILE, 128), jnp.float32),
        scratch_types=[
            pltpu.VMEM((HIST, B_PER_W), jnp.int32),
            pltpu.VMEM((NBUF, T_CHUNK * B_PER_W, FEAT), jnp.float32),
            pltpu.VMEM((NBUF, T_CHUNK, NFT, FTILE, 128), jnp.float32),
            pltpu.SemaphoreType.DMA((NBUF,)),
            pltpu.SemaphoreType.DMA((NBUF,)),
        ],
        compiler_params=pltpu.CompilerParams(
            use_tc_tiling_on_sc=False, needs_layout_passes=False
        ),
    )(_embed_kernel)
    return k(idx_t, table)


def kernel(inputs, embedding):
    out4 = _embed(inputs.T.astype(jnp.int32), embedding)
    return jnp.reshape(jnp.transpose(out4, (2, 4, 0, 1, 3)), (BATCH, HIST, FEAT))


# speedup vs baseline: 1.0292x; 1.0292x over previous
import functools

import jax
import jax.numpy as jnp
from jax import lax
from jax.experimental import pallas as pl
from jax.experimental.pallas import tpu as pltpu
from jax.experimental.pallas import tpu_sc as plsc

BATCH = 4096
HIST = 200
FEAT = 32

NUM_CORES = 2
NUM_SUBCORES = 16
NW = NUM_CORES * NUM_SUBCORES
B_PER_W = BATCH // NW
NBUF = 8
LANES = 16
FTILE = 8
NFT = FEAT // FTILE


def _embed_kernel(idx_hbm, table_hbm, out_hbm, idx_tv, bufs, obufs, gsem, osem):
    wid = lax.axis_index("s") * NUM_CORES + lax.axis_index("c")
    b0 = wid * B_PER_W
    pltpu.sync_copy(idx_hbm.at[:, pl.ds(b0, B_PER_W)], idx_tv)

    lane_iota = lax.iota(jnp.int32, LANES)

    def gather(t, s):
        return pltpu.make_async_copy(
            table_hbm.at[idx_tv.at[t]], bufs.at[s], gsem.at[s]
        )

    def out_copy(t, s):
        return pltpu.make_async_copy(
            obufs.at[s], out_hbm.at[t, :, wid], osem.at[s]
        )

    for t in range(NBUF - 1):
        gather(t, t).start()

    def body(t, carry):
        s = t % NBUF
        s_ids = jnp.full((LANES,), s, jnp.int32)

        @pl.when(t + NBUF - 1 < HIST)
        def _():
            gather(t + NBUF - 1, (t + NBUF - 1) % NBUF).start()

        gather(t, s).wait()

        @pl.when(t >= NBUF)
        def _():
            out_copy(t - NBUF, s).wait()

        kgroups = B_PER_W // LANES

        @plsc.parallel_loop(0, FEAT * kgroups, unroll=16)
        def _(i):
            f = i // kgroups
            k = i % kgroups
            v = plsc.load_gather(
                bufs,
                [s_ids, k * LANES + lane_iota, jnp.full((LANES,), f, jnp.int32)],
            )
            obufs[s, f // FTILE, f % FTILE, pl.ds(k * LANES, LANES)] = v

        out_copy(t, s).start()
        return carry

    lax.fori_loop(0, HIST, body, None)
    for t in range(HIST - NBUF, HIST):
        out_copy(t, t % NBUF).wait()


@jax.jit
def _embed(idx_t, table):
    k = functools.partial(
        pl.kernel,
        mesh=plsc.VectorSubcoreMesh(core_axis_name="c", subcore_axis_name="s"),
        out_type=jax.ShapeDtypeStruct(
            (HIST, NFT, NW, FTILE, 128), jnp.float32
        ),
        scratch_types=[
            pltpu.VMEM((HIST, B_PER_W), jnp.int32),
            pltpu.VMEM((NBUF, B_PER_W, FEAT), jnp.float32),
            pltpu.VMEM((NBUF, NFT, FTILE, B_PER_W), jnp.float32),
            pltpu.SemaphoreType.DMA((NBUF,)),
            pltpu.SemaphoreType.DMA((NBUF,)),
        ],
        compiler_params=pltpu.CompilerParams(
            use_tc_tiling_on_sc=False, needs_layout_passes=False
        ),
    )(_embed_kernel)
    return k(idx_t, table)


def kernel(inputs, embedding):
    out4 = _embed(inputs.T.astype(jnp.int32), embedding)
    return jnp.reshape(jnp.transpose(out4, (2, 4, 0, 1, 3)), (BATCH, HIST, FEAT))


# speedup vs baseline: 1.0416x; 1.0121x over previous
import functools

import jax
import jax.numpy as jnp
from jax import lax
from jax.experimental import pallas as pl
from jax.experimental.pallas import tpu as pltpu
from jax.experimental.pallas import tpu_sc as plsc

BATCH = 4096
HIST = 200
FEAT = 32

NUM_CORES = 2
NUM_SUBCORES = 16
NW = NUM_CORES * NUM_SUBCORES
B_PER_W = BATCH // NW
NBUF = 8
LANES = 16
FTILE = 8
NFT = FEAT // FTILE


def _embed_kernel(idx_hbm, table_hbm, out_hbm, idx_tv, bufs, obufs, gsem, osem):
    wid = lax.axis_index("s") * NUM_CORES + lax.axis_index("c")
    b0 = wid * B_PER_W
    pltpu.sync_copy(idx_hbm.at[:, pl.ds(b0, B_PER_W)], idx_tv)

    lane_iota = lax.iota(jnp.int32, LANES)

    def gather(t, s):
        return pltpu.make_async_copy(
            table_hbm.at[idx_tv.at[t]], bufs.at[s], gsem.at[s]
        )

    def out_copy(t, s):
        return pltpu.make_async_copy(
            obufs.at[s], out_hbm.at[t, :, wid], osem.at[s]
        )

    for t in range(NBUF - 1):
        gather(t, t).start()

    def body(t, carry):
        s = t % NBUF
        s_ids = jnp.full((LANES,), s, jnp.int32)

        @pl.when(t + NBUF - 1 < HIST)
        def _():
            gather(t + NBUF - 1, (t + NBUF - 1) % NBUF).start()

        gather(t, s).wait()

        @pl.when(t >= NBUF)
        def _():
            out_copy(t - NBUF, s).wait()

        kgroups = B_PER_W // LANES

        @plsc.parallel_loop(0, FEAT * kgroups, unroll=32)
        def _(i):
            f = i // kgroups
            k = i % kgroups
            v = plsc.load_gather(
                bufs,
                [s_ids, k * LANES + lane_iota, jnp.full((LANES,), f, jnp.int32)],
            )
            obufs[s, f // FTILE, f % FTILE, pl.ds(k * LANES, LANES)] = v

        out_copy(t, s).start()
        return carry

    lax.fori_loop(0, HIST, body, None)
    for t in range(HIST - NBUF, HIST):
        out_copy(t, t % NBUF).wait()


@jax.jit
def _embed(idx_t, table):
    k = functools.partial(
        pl.kernel,
        mesh=plsc.VectorSubcoreMesh(core_axis_name="c", subcore_axis_name="s"),
        out_type=jax.ShapeDtypeStruct(
            (HIST, NFT, NW, FTILE, 128), jnp.float32
        ),
        scratch_types=[
            pltpu.VMEM((HIST, B_PER_W), jnp.int32),
            pltpu.VMEM((NBUF, B_PER_W, FEAT), jnp.float32),
            pltpu.VMEM((NBUF, NFT, FTILE, B_PER_W), jnp.float32),
            pltpu.SemaphoreType.DMA((NBUF,)),
            pltpu.SemaphoreType.DMA((NBUF,)),
        ],
        compiler_params=pltpu.CompilerParams(
            use_tc_tiling_on_sc=False, needs_layout_passes=False
        ),
    )(_embed_kernel)
    return k(idx_t, table)


def kernel(inputs, embedding):
    out4 = _embed(inputs.T.astype(jnp.int32), embedding)
    return jnp.reshape(jnp.transpose(out4, (2, 4, 0, 1, 3)), (BATCH, HIST, FEAT))
